# Initial kernel scaffold; baseline (speedup 1.0000x reference)
import functools

import jax
import jax.numpy as jnp
from jax import lax
from jax.experimental import pallas as pl
from jax.experimental.pallas import tpu as pltpu
from jax.experimental.pallas import tpu_sc as plsc

N_CORES = 2
N_SUBCORES = 16
L = 16

B = 16384
F = 100
M = 49
D = 32

FH = F // 2
BW = B // N_SUBCORES
NB = 8
NP = NB * FH
NCHUNK = BW // NB
ROW_W = M * D


def _body(x_hbm, tab_hbm, e0_hbm, hinv_hbm, out_hbm,
          tab_v, xb, e0_v, hinv_v, e0p, hip, fbase, rowp, colg, offb, tb, outb):
    cid = lax.axis_index("c")
    sid = lax.axis_index("s")
    f0 = cid * FH
    bw0 = sid * BW

    pltpu.sync_copy(tab_hbm.at[pl.ds(f0 * ROW_W, FH * ROW_W)], tab_v)
    pltpu.sync_copy(e0_hbm, e0_v)
    pltpu.sync_copy(hinv_hbm, hinv_v)

    def pat(i, _):
        pv = lax.iota(jnp.int32, L) + i * L
        col = pv % FH
        row = pv // FH
        fg = col + f0
        sl = pl.ds(i * L, L)
        e0p[sl] = plsc.load_gather(e0_v, [fg])
        hip[sl] = plsc.load_gather(hinv_v, [fg])
        fbase[sl] = col * ROW_W
        rowp[sl] = row
        colg[sl] = fg
        return 0

    lax.fori_loop(0, NP // L, pat, 0)

    def chunk(c, _):
        b0 = bw0 + c * NB
        pltpu.sync_copy(x_hbm.at[pl.ds(b0, NB)], xb)

        def pre(i, _):
            sl = pl.ds(i * L, L)
            xv = plsc.load_gather(xb, [rowp[sl], colg[sl]])
            v = (xv - e0p[sl]) * hip[sl]
            bn = v.astype(jnp.int32)
            bn = jnp.minimum(jnp.maximum(bn, 0), M - 2)
            t = jnp.clip(v - bn.astype(jnp.float32), 0.0, 1.0)
            offb[sl] = fbase[sl] + bn * D
            tb[sl] = t
            return 0

        lax.fori_loop(0, NP // L, pre, 0, unroll=4)

        for r in range(NB):
            def lerp(cc, _, r=r):
                p = r * FH + cc
                off = offb[p]
                tv = lax.broadcast_in_dim(tb[p], (L,), ())
                l0 = tab_v[pl.ds(off, L)]
                l1 = tab_v[pl.ds(off + L, L)]
                r0 = tab_v[pl.ds(off + D, L)]
                r1 = tab_v[pl.ds(off + D + L, L)]
                outb[r, cc, pl.ds(0, L)] = l0 + tv * (r0 - l0)
                outb[r, cc, pl.ds(L, L)] = l1 + tv * (r1 - l1)
                return 0

            lax.fori_loop(0, FH, lerp, 0, unroll=4)

        pltpu.sync_copy(outb, out_hbm.at[pl.ds(b0, NB), pl.ds(f0, FH), :])
        return 0

    lax.fori_loop(0, NCHUNK, chunk, 0)


@jax.jit
def kernel(x, bin_edges, boundary_embeddings):
    e0 = bin_edges[:, 0]
    h = bin_edges[:, 1] - bin_edges[:, 0]
    hinv = jnp.where(jnp.abs(h) < 1e-8, 1.0, 1.0 / h)
    tab_flat = boundary_embeddings.reshape(F * M * D)

    mesh = plsc.VectorSubcoreMesh(core_axis_name="c", subcore_axis_name="s")
    run = pl.kernel(
        _body,
        out_type=jax.ShapeDtypeStruct((B, F, D), jnp.float32),
        mesh=mesh,
        scratch_types=[
            pltpu.VMEM((FH * ROW_W,), jnp.float32),
            pltpu.VMEM((NB, F), jnp.float32),
            pltpu.VMEM((F,), jnp.float32),
            pltpu.VMEM((F,), jnp.float32),
            pltpu.VMEM((NP,), jnp.float32),
            pltpu.VMEM((NP,), jnp.float32),
            pltpu.VMEM((NP,), jnp.int32),
            pltpu.VMEM((NP,), jnp.int32),
            pltpu.VMEM((NP,), jnp.int32),
            pltpu.VMEM((NP,), jnp.int32),
            pltpu.VMEM((NP,), jnp.float32),
            pltpu.VMEM((NB, FH, D), jnp.float32),
        ],
    )
    return run(x, tab_flat, e0, hinv)


# speedup vs baseline: 95.9896x; 95.9896x over previous
import jax
import jax.numpy as jnp
from jax import lax
from jax.experimental import pallas as pl
from jax.experimental.pallas import tpu as pltpu
from jax.experimental.pallas import tpu_sc as plsc

N_CORES = 2
N_SUBCORES = 16
L = 16

B = 16384
F = 100
M = 49
D = 32

FH = F // 2
BW = B // N_SUBCORES
NB = 8
NP = NB * FH
NCHUNK = BW // NB


def _splat(s):
    return lax.broadcast_in_dim(s, (L,), ())


def _body(x_hbm, tab_hbm, e0_hbm, hinv_hbm, out_hbm,
          tab_v, xb, e0_v, hinv_v, e0p, hip, fbase, rowp, colg, offb, tb, outb):
    cid = lax.axis_index("c")
    sid = lax.axis_index("s")
    f0 = cid * FH
    bw0 = sid * BW

    pltpu.sync_copy(tab_hbm.at[cid], tab_v)
    pltpu.sync_copy(e0_hbm, e0_v)
    pltpu.sync_copy(hinv_hbm, hinv_v)

    f0v = _splat(f0)

    def pat(i, cr):
        col, row = cr
        fg = col + f0v
        sl = pl.ds(i * L, L)
        e0p[sl] = plsc.load_gather(e0_v, [fg])
        hip[sl] = plsc.load_gather(hinv_v, [fg])
        fbase[sl] = col * M
        rowp[sl] = row
        colg[sl] = fg
        ncol = col + L
        wrap = ncol >= FH
        ncol = jnp.where(wrap, ncol - FH, ncol)
        nrow = jnp.where(wrap, row + 1, row)
        return ncol, nrow

    col0 = lax.iota(jnp.int32, L)
    row0 = jnp.zeros((L,), jnp.int32)
    lax.fori_loop(0, NP // L, pat, (col0, row0))

    def chunk(c, _):
        b0 = bw0 + c * NB
        pltpu.sync_copy(x_hbm.at[pl.ds(b0, NB)], xb)

        def pre(i, _):
            sl = pl.ds(i * L, L)
            xv = plsc.load_gather(xb, [rowp[sl], colg[sl]])
            v = (xv - e0p[sl]) * hip[sl]
            bn = v.astype(jnp.int32)
            bn = jnp.minimum(jnp.maximum(bn, 0), M - 2)
            t = jnp.clip(v - bn.astype(jnp.float32), 0.0, 1.0)
            offb[sl] = fbase[sl] + bn
            tb[sl] = t
            return 0

        lax.fori_loop(0, NP // L, pre, 0)

        def lerp(i, _):
            sl = pl.ds(i * L, L)
            ov = offb[sl]
            tvv = tb[sl]
            rv = rowp[sl]
            cv = colg[sl]
            for k in range(L):
                off = ov[k]
                r = rv[k]
                cc = cv[k] - f0
                tv = _splat(tvv[k])
                l0 = tab_v[off, pl.ds(0, L)]
                l1 = tab_v[off, pl.ds(L, L)]
                r0 = tab_v[off + 1, pl.ds(0, L)]
                r1 = tab_v[off + 1, pl.ds(L, L)]
                outb[r, cc, pl.ds(0, L)] = l0 + tv * (r0 - l0)
                outb[r, cc, pl.ds(L, L)] = l1 + tv * (r1 - l1)
            return 0

        lax.fori_loop(0, NP // L, lerp, 0)

        pltpu.sync_copy(outb, out_hbm.at[pl.ds(b0, NB), cid])
        return 0

    lax.fori_loop(0, NCHUNK, chunk, 0)


@jax.jit
def kernel(x, bin_edges, boundary_embeddings):
    e0 = bin_edges[:, 0]
    h = bin_edges[:, 1] - bin_edges[:, 0]
    hinv = jnp.where(jnp.abs(h) < 1e-8, 1.0, 1.0 / h)
    tab3 = boundary_embeddings.reshape(N_CORES, FH * M, D)

    mesh = plsc.VectorSubcoreMesh(core_axis_name="c", subcore_axis_name="s")
    run = pl.kernel(
        _body,
        out_type=jax.ShapeDtypeStruct((B, N_CORES, FH, D), jnp.float32),
        mesh=mesh,
        compiler_params=pltpu.CompilerParams(
            use_tc_tiling_on_sc=False, needs_layout_passes=False),
        scratch_types=[
            pltpu.VMEM((FH * M, D), jnp.float32),
            pltpu.VMEM((NB, F), jnp.float32),
            pltpu.VMEM((F,), jnp.float32),
            pltpu.VMEM((F,), jnp.float32),
            pltpu.VMEM((NP,), jnp.float32),
            pltpu.VMEM((NP,), jnp.float32),
            pltpu.VMEM((NP,), jnp.int32),
            pltpu.VMEM((NP,), jnp.int32),
            pltpu.VMEM((NP,), jnp.int32),
            pltpu.VMEM((NP,), jnp.int32),
            pltpu.VMEM((NP,), jnp.float32),
            pltpu.VMEM((NB, FH, D), jnp.float32),
        ],
    )
    out = run(x, tab3, e0, hinv)
    return out.reshape(B, F, D)
